# Initial kernel scaffold; baseline (speedup 1.0000x reference)
import jax
import jax.numpy as jnp
from jax.experimental import pallas as pl


def kernel(x, edge_index, diameter, enc_W, enc_b, proc_W, proc_b, dec_W, dec_b):
    raise NotImplementedError("write your pallas kernel here")


# speedup vs baseline: 1.9729x; 1.9729x over previous
import functools

import jax
import jax.numpy as jnp
from jax import lax
from jax.experimental import pallas as pl
from jax.experimental.pallas import tpu as pltpu
from jax.experimental.pallas import tpu_sc as plsc

NSUB = 32
LANES = 16
CH = 3200


def _round_up(v, m):
    return (v + m - 1) // m * m


def _make_encoder(NPAD, F, H):
    def body(x_ref, w_ref, b_ref, out_ref):
        xb = x_ref[...]
        w = w_ref[...]
        out_ref[...] = (
            lax.dot_general(w, xb, (((1,), (1,)), ((), ())),
                            preferred_element_type=jnp.float32)
            + b_ref[...]
        )

    return pl.pallas_call(
        body,
        out_shape=jax.ShapeDtypeStruct((H, NPAD), jnp.float32),
    )


def _make_mlp(NPAD, H):
    def body(h_ref, agg_ref, w_ref, b_ref, out_ref):
        s = h_ref[...] + agg_ref[...]
        out_ref[...] = (
            lax.dot_general(w_ref[...], s, (((1,), (0,)), ((), ())),
                            preferred_element_type=jnp.float32)
            + b_ref[...]
        )

    return pl.pallas_call(
        body,
        out_shape=jax.ShapeDtypeStruct((H, NPAD), jnp.float32),
    )


def _make_decoder(NPAD, H, C):
    def body(h_ref, w0_ref, b0_ref, w1_ref, b1_ref, o0_ref, o1_ref):
        h = h_ref[...]
        for w_ref, b_ref, o_ref in ((w0_ref, b0_ref, o0_ref),
                                    (w1_ref, b1_ref, o1_ref)):
            logits = lax.dot_general(
                w_ref[...], h, (((1,), (0,)), ((), ())),
                preferred_element_type=jnp.float32) + b_ref[...]
            m = jnp.max(logits, axis=0, keepdims=True)
            e = jnp.exp(logits - m)
            s = jnp.sum(e, axis=0, keepdims=True)
            o_ref[...] = logits - m - jnp.log(s)

    return pl.pallas_call(
        body,
        out_shape=(jax.ShapeDtypeStruct((C, NPAD), jnp.float32),
                   jax.ShapeDtypeStruct((C, NPAD), jnp.float32)),
    )


def _make_segmax(NPAD, H, E_pad):
    FPT = H // NSUB
    nchunk = E_pad // CH
    nbatch = CH // LANES
    ninit = FPT * NPAD // LANES
    ncnt = NPAD // LANES

    mesh = plsc.VectorSubcoreMesh(core_axis_name="c", subcore_axis_name="s")

    @functools.partial(
        pl.kernel,
        out_type=jax.ShapeDtypeStruct((H, NPAD), jnp.float32),
        mesh=mesh,
        compiler_params=pltpu.CompilerParams(
            needs_layout_passes=False, use_tc_tiling_on_sc=False),
        scratch_types=[
            pltpu.VMEM((FPT * NPAD,), jnp.float32),
            pltpu.VMEM((FPT * NPAD,), jnp.float32),
            pltpu.VMEM((NPAD,), jnp.int32),
            pltpu.VMEM((2, 2, CH), jnp.int32),
            pltpu.SemaphoreType.DMA,
            pltpu.SemaphoreType.DMA,
        ],
    )
    def segmax(hT, edges, aggT, h_l, agg_l, cnt, eb, semA, semB):
        c = lax.axis_index("c")
        s = lax.axis_index("s")
        wid = s * 2 + c
        fbase = wid * FPT
        sems = (semA, semB)

        for k in range(FPT):
            pltpu.sync_copy(hT.at[fbase + k], h_l.at[pl.ds(k * NPAD, NPAD)])

        neg = jnp.full((LANES,), -jnp.inf, jnp.float32)
        zeros = jnp.zeros((LANES,), jnp.int32)

        def init_agg(i, carry):
            agg_l[pl.ds(i * LANES, LANES)] = neg
            return carry

        lax.fori_loop(0, ninit, init_agg, 0)

        def init_cnt(i, carry):
            cnt[pl.ds(i * LANES, LANES)] = zeros
            return carry

        lax.fori_loop(0, ncnt, init_cnt, 0)

        lane = lax.iota(jnp.int32, LANES)
        bit = jnp.left_shift(jnp.ones((LANES,), jnp.int32), lane)
        full_mask = lane < LANES

        def start(b, ci):
            off = ci * CH
            pltpu.make_async_copy(edges.at[0, pl.ds(off, CH)],
                                  eb.at[b, 0], sems[b]).start()
            pltpu.make_async_copy(edges.at[1, pl.ds(off, CH)],
                                  eb.at[b, 1], sems[b]).start()

        def wait(b):
            pltpu.make_async_copy(edges.at[0, pl.ds(0, CH)],
                                  eb.at[b, 0], sems[b]).wait()
            pltpu.make_async_copy(edges.at[1, pl.ds(0, CH)],
                                  eb.at[b, 1], sems[b]).wait()

        def process(b):
            def batch(t, carry):
                base = t * LANES
                srcv = eb[b, 0, pl.ds(base, LANES)]
                dstv = eb[b, 1, pl.ds(base, LANES)]
                msgs = [plsc.load_gather(h_l, [srcv + k * NPAD])
                        for k in range(FPT)]
                didx = [dstv + k * NPAD for k in range(FPT)]

                def cond(rem):
                    return jnp.any(rem)

                def round_body(rem):
                    plsc.addupdate_scatter(cnt, [dstv], bit, mask=rem)
                    g = plsc.load_gather(cnt, [dstv])
                    plsc.addupdate_scatter(cnt, [dstv], -bit, mask=rem)
                    w = rem & ((g & (-g)) == bit)
                    for k in range(FPT):
                        cur = plsc.load_gather(agg_l, [didx[k]])
                        plsc.store_scatter(agg_l, [didx[k]],
                                           jnp.maximum(cur, msgs[k]), mask=w)
                    return rem & jnp.logical_not(w)

                lax.while_loop(cond, round_body, full_mask)
                return carry

            lax.fori_loop(0, nbatch, batch, 0)

        start(0, 0)
        start(1, min(1, nchunk - 1))

        def chunk_iter(j, carry):
            for b in range(2):
                ci = j * 2 + b
                wait(b)
                process(b)

                @pl.when(ci + 2 < nchunk)
                def _():
                    start(b, ci + 2)

            return carry

        lax.fori_loop(0, nchunk // 2, chunk_iter, 0)

        def finalize(i, carry):
            v = agg_l[pl.ds(i * LANES, LANES)]
            agg_l[pl.ds(i * LANES, LANES)] = jnp.where(
                v == -jnp.inf, jnp.zeros((LANES,), jnp.float32), v)
            return carry

        lax.fori_loop(0, ninit, finalize, 0)

        for k in range(FPT):
            pltpu.sync_copy(agg_l.at[pl.ds(k * NPAD, NPAD)], aggT.at[fbase + k])

    return segmax


def kernel(x, edge_index, diameter, enc_W, enc_b, proc_W, proc_b, dec_W, dec_b):
    N, F = x.shape
    H = enc_W.shape[0]
    P, C = dec_b.shape
    E = edge_index.shape[1]

    NPAD = _round_up(N, LANES)
    E_pad = _round_up(E, 2 * CH)
    if NPAD == N and E_pad != E:
        NPAD += LANES
    x_p = jnp.pad(x, ((0, NPAD - N), (0, 0))) if NPAD != N else x
    if E_pad != E:
        pad = jnp.concatenate(
            [jnp.zeros((1, E_pad - E), jnp.int32),
             jnp.full((1, E_pad - E), N, jnp.int32)], axis=0)
        edges = jnp.concatenate([edge_index, pad], axis=1)
    else:
        edges = edge_index

    encoder = _make_encoder(NPAD, F, H)
    mlp = _make_mlp(NPAD, H)
    segmax = _make_segmax(NPAD, H, E_pad)
    decoder = _make_decoder(NPAD, H, C)

    hT = encoder(x_p, enc_W, enc_b[:, None])

    def body(_, hT):
        aggT = segmax(hT, edges)
        return mlp(hT, aggT, proc_W, proc_b[:, None])

    hT = lax.fori_loop(0, diameter, body, hT)

    o0, o1 = decoder(hT, dec_W[0], dec_b[0][:, None], dec_W[1], dec_b[1][:, None])
    return (o0.T[:N], o1.T[:N])


# speedup vs baseline: 2.0833x; 1.0559x over previous
import functools

import jax
import jax.numpy as jnp
from jax import lax
from jax.experimental import pallas as pl
from jax.experimental.pallas import tpu as pltpu
from jax.experimental.pallas import tpu_sc as plsc

NSUB = 32
LANES = 16
CH = 3200


def _round_up(v, m):
    return (v + m - 1) // m * m


def _make_encoder(NPAD, F, H):
    def body(x_ref, w_ref, b_ref, out_ref):
        xb = x_ref[...]
        w = w_ref[...]
        out_ref[...] = (
            lax.dot_general(w, xb, (((1,), (1,)), ((), ())),
                            preferred_element_type=jnp.float32)
            + b_ref[...]
        )

    return pl.pallas_call(
        body,
        out_shape=jax.ShapeDtypeStruct((H, NPAD), jnp.float32),
    )


def _make_mlp(NPAD, H):
    def body(h_ref, agg_ref, w_ref, b_ref, out_ref):
        s = h_ref[...] + agg_ref[...]
        out_ref[...] = (
            lax.dot_general(w_ref[...], s, (((1,), (0,)), ((), ())),
                            preferred_element_type=jnp.float32)
            + b_ref[...]
        )

    return pl.pallas_call(
        body,
        out_shape=jax.ShapeDtypeStruct((H, NPAD), jnp.float32),
    )


def _make_decoder(NPAD, H, C):
    def body(h_ref, w0_ref, b0_ref, w1_ref, b1_ref, o0_ref, o1_ref):
        h = h_ref[...]
        for w_ref, b_ref, o_ref in ((w0_ref, b0_ref, o0_ref),
                                    (w1_ref, b1_ref, o1_ref)):
            logits = lax.dot_general(
                w_ref[...], h, (((1,), (0,)), ((), ())),
                preferred_element_type=jnp.float32) + b_ref[...]
            m = jnp.max(logits, axis=0, keepdims=True)
            e = jnp.exp(logits - m)
            s = jnp.sum(e, axis=0, keepdims=True)
            o_ref[...] = logits - m - jnp.log(s)

    return pl.pallas_call(
        body,
        out_shape=(jax.ShapeDtypeStruct((C, NPAD), jnp.float32),
                   jax.ShapeDtypeStruct((C, NPAD), jnp.float32)),
    )


def _make_segmax(NPAD, H, E_pad):
    FPT = H // NSUB
    nchunk = E_pad // CH
    nbatch = CH // LANES
    ninit = FPT * NPAD // LANES
    ncnt = NPAD // LANES

    mesh = plsc.VectorSubcoreMesh(core_axis_name="c", subcore_axis_name="s")

    @functools.partial(
        pl.kernel,
        out_type=jax.ShapeDtypeStruct((H, NPAD), jnp.float32),
        mesh=mesh,
        compiler_params=pltpu.CompilerParams(
            needs_layout_passes=False, use_tc_tiling_on_sc=False),
        scratch_types=[
            [pltpu.VMEM((NPAD,), jnp.float32) for _ in range(FPT)],
            [pltpu.VMEM((NPAD,), jnp.float32) for _ in range(FPT)],
            pltpu.VMEM((2, 2, CH), jnp.int32),
            pltpu.SemaphoreType.DMA,
            pltpu.SemaphoreType.DMA,
        ],
    )
    def segmax(hT, edges, aggT, hbufs, abufs, eb, semA, semB):
        c = lax.axis_index("c")
        s = lax.axis_index("s")
        wid = s * 2 + c
        fbase = wid * FPT
        sems = (semA, semB)

        for k in range(FPT):
            pltpu.sync_copy(hT.at[fbase + k], hbufs[k])

        neg = jnp.full((LANES,), -jnp.inf, jnp.float32)

        def init_agg(i, carry):
            for k in range(FPT):
                abufs[k][pl.ds(i * LANES, LANES)] = neg
            return carry

        lax.fori_loop(0, ncnt, init_agg, 0)

        def start(b, ci):
            off = ci * CH
            pltpu.make_async_copy(edges.at[0, pl.ds(off, CH)],
                                  eb.at[b, 0], sems[b]).start()
            pltpu.make_async_copy(edges.at[1, pl.ds(off, CH)],
                                  eb.at[b, 1], sems[b]).start()

        def wait(b):
            pltpu.make_async_copy(edges.at[0, pl.ds(0, CH)],
                                  eb.at[b, 0], sems[b]).wait()
            pltpu.make_async_copy(edges.at[1, pl.ds(0, CH)],
                                  eb.at[b, 1], sems[b]).wait()

        def process(b):
            def one_batch(base):
                srcv = eb[b, 0, pl.ds(base, LANES)]
                dstv = eb[b, 1, pl.ds(base, LANES)]
                msgs = [plsc.load_gather(hbufs[k], [srcv])
                        for k in range(FPT)]
                cnt, _ = plsc.scan_count(dstv)

                def apply_round(w):
                    curs = [plsc.load_gather(abufs[k], [dstv])
                            for k in range(FPT)]
                    news = [jnp.maximum(curs[k], msgs[k])
                            for k in range(FPT)]
                    for k in range(FPT):
                        plsc.store_scatter(abufs[k], [dstv], news[k], mask=w)

                cmax = jnp.max(cnt)
                apply_round(cnt == 0)

                @pl.when(cmax > 0)
                def _():
                    def round_body(r, carry):
                        apply_round(cnt == r)
                        return carry

                    lax.fori_loop(1, cmax + 1, round_body, 0)

            def batch2(t, carry):
                one_batch(t * (2 * LANES))
                one_batch(t * (2 * LANES) + LANES)
                return carry

            lax.fori_loop(0, nbatch // 2, batch2, 0)

        start(0, 0)
        start(1, min(1, nchunk - 1))

        def chunk_iter(j, carry):
            for b in range(2):
                ci = j * 2 + b
                wait(b)
                process(b)

                @pl.when(ci + 2 < nchunk)
                def _():
                    start(b, ci + 2)

            return carry

        lax.fori_loop(0, nchunk // 2, chunk_iter, 0)

        zero16 = jnp.zeros((LANES,), jnp.float32)

        def finalize(i, carry):
            for k in range(FPT):
                v = abufs[k][pl.ds(i * LANES, LANES)]
                abufs[k][pl.ds(i * LANES, LANES)] = jnp.where(
                    v == -jnp.inf, zero16, v)
            return carry

        lax.fori_loop(0, ncnt, finalize, 0)

        for k in range(FPT):
            pltpu.sync_copy(abufs[k], aggT.at[fbase + k])

    return segmax


def kernel(x, edge_index, diameter, enc_W, enc_b, proc_W, proc_b, dec_W, dec_b):
    N, F = x.shape
    H = enc_W.shape[0]
    P, C = dec_b.shape
    E = edge_index.shape[1]

    NPAD = _round_up(N, LANES)
    E_pad = _round_up(E, 2 * CH)
    if NPAD == N and E_pad != E:
        NPAD += LANES
    x_p = jnp.pad(x, ((0, NPAD - N), (0, 0))) if NPAD != N else x
    if E_pad != E:
        pad = jnp.concatenate(
            [jnp.zeros((1, E_pad - E), jnp.int32),
             jnp.full((1, E_pad - E), N, jnp.int32)], axis=0)
        edges = jnp.concatenate([edge_index, pad], axis=1)
    else:
        edges = edge_index

    encoder = _make_encoder(NPAD, F, H)
    mlp = _make_mlp(NPAD, H)
    segmax = _make_segmax(NPAD, H, E_pad)
    decoder = _make_decoder(NPAD, H, C)

    hT = encoder(x_p, enc_W, enc_b[:, None])

    def body(_, hT):
        aggT = segmax(hT, edges)
        return mlp(hT, aggT, proc_W, proc_b[:, None])

    hT = lax.fori_loop(0, diameter, body, hT)

    o0, o1 = decoder(hT, dec_W[0], dec_b[0][:, None], dec_W[1], dec_b[1][:, None])
    return (o0.T[:N], o1.T[:N])


# speedup vs baseline: 2.2131x; 1.0623x over previous
import functools

import jax
import jax.numpy as jnp
from jax import lax
from jax.experimental import pallas as pl
from jax.experimental.pallas import tpu as pltpu
from jax.experimental.pallas import tpu_sc as plsc

NSUB = 32
LANES = 16
CH = 3200


def _round_up(v, m):
    return (v + m - 1) // m * m


def _make_encoder(NPAD, F, H):
    def body(x_ref, w_ref, b_ref, out_ref):
        xb = x_ref[...]
        w = w_ref[...]
        out_ref[...] = (
            lax.dot_general(w, xb, (((1,), (1,)), ((), ())),
                            preferred_element_type=jnp.float32)
            + b_ref[...]
        )

    return pl.pallas_call(
        body,
        out_shape=jax.ShapeDtypeStruct((H, NPAD), jnp.float32),
    )


def _make_mlp(NPAD, H):
    def body(h_ref, agg_ref, w_ref, b_ref, out_ref):
        s = h_ref[...] + agg_ref[...]
        out_ref[...] = (
            lax.dot_general(w_ref[...], s, (((1,), (0,)), ((), ())),
                            preferred_element_type=jnp.float32)
            + b_ref[...]
        )

    return pl.pallas_call(
        body,
        out_shape=jax.ShapeDtypeStruct((H, NPAD), jnp.float32),
    )


def _make_decoder(NPAD, H, C):
    def body(h_ref, w0_ref, b0_ref, w1_ref, b1_ref, o0_ref, o1_ref):
        h = h_ref[...]
        for w_ref, b_ref, o_ref in ((w0_ref, b0_ref, o0_ref),
                                    (w1_ref, b1_ref, o1_ref)):
            logits = lax.dot_general(
                w_ref[...], h, (((1,), (0,)), ((), ())),
                preferred_element_type=jnp.float32) + b_ref[...]
            m = jnp.max(logits, axis=0, keepdims=True)
            e = jnp.exp(logits - m)
            s = jnp.sum(e, axis=0, keepdims=True)
            o_ref[...] = logits - m - jnp.log(s)

    return pl.pallas_call(
        body,
        out_shape=(jax.ShapeDtypeStruct((C, NPAD), jnp.float32),
                   jax.ShapeDtypeStruct((C, NPAD), jnp.float32)),
    )


def _make_segmax(NPAD, H, E_pad):
    FPT = H // NSUB
    nchunk = E_pad // CH
    nbatch = CH // LANES
    ninit = FPT * NPAD // LANES
    ncnt = NPAD // LANES

    mesh = plsc.VectorSubcoreMesh(core_axis_name="c", subcore_axis_name="s")

    @functools.partial(
        pl.kernel,
        out_type=jax.ShapeDtypeStruct((H, NPAD), jnp.float32),
        mesh=mesh,
        compiler_params=pltpu.CompilerParams(
            needs_layout_passes=False, use_tc_tiling_on_sc=False),
        scratch_types=[
            [pltpu.VMEM((NPAD,), jnp.float32) for _ in range(FPT)],
            [pltpu.VMEM((NPAD,), jnp.float32) for _ in range(FPT)],
            pltpu.VMEM((2, 2, CH), jnp.int32),
            pltpu.SemaphoreType.DMA,
            pltpu.SemaphoreType.DMA,
        ],
    )
    def segmax(hT, edges, aggT, hbufs, abufs, eb, semA, semB):
        c = lax.axis_index("c")
        s = lax.axis_index("s")
        wid = s * 2 + c
        fbase = wid * FPT
        sems = (semA, semB)

        for k in range(FPT):
            pltpu.make_async_copy(hT.at[fbase + k], hbufs[k], semA).start()

        neg = jnp.full((LANES,), -jnp.inf, jnp.float32)

        def init_agg(i, carry):
            for k in range(FPT):
                abufs[k][pl.ds(i * LANES, LANES)] = neg
            return carry

        lax.fori_loop(0, ncnt, init_agg, 0)

        for k in range(FPT):
            pltpu.make_async_copy(hT.at[fbase + k], hbufs[k], semA).wait()

        def start(b, ci):
            off = ci * CH
            pltpu.make_async_copy(edges.at[0, pl.ds(off, CH)],
                                  eb.at[b, 0], sems[b]).start()
            pltpu.make_async_copy(edges.at[1, pl.ds(off, CH)],
                                  eb.at[b, 1], sems[b]).start()

        def wait(b):
            pltpu.make_async_copy(edges.at[0, pl.ds(0, CH)],
                                  eb.at[b, 0], sems[b]).wait()
            pltpu.make_async_copy(edges.at[1, pl.ds(0, CH)],
                                  eb.at[b, 1], sems[b]).wait()

        def fetch(b, base):
            srcv = eb[b, 0, pl.ds(base, LANES)]
            dstv = eb[b, 1, pl.ds(base, LANES)]
            msgs = [plsc.load_gather(hbufs[k], [srcv]) for k in range(FPT)]
            cnt, _ = plsc.scan_count(dstv)
            cmax = jnp.max(cnt)
            return (dstv, cnt, cmax, *msgs)

        def apply_round(dstv, msgs, w):
            curs = [plsc.load_gather(abufs[k], [dstv]) for k in range(FPT)]
            news = [jnp.maximum(curs[k], msgs[k]) for k in range(FPT)]
            for k in range(FPT):
                plsc.store_scatter(abufs[k], [dstv], news[k], mask=w)

        def process(b):
            def batch(t, carry):
                dstv, cnt, cmax, *msgs = carry
                nxt = fetch(b, jnp.minimum(t + 1, nbatch - 1) * LANES)
                apply_round(dstv, msgs, cnt == 0)

                @pl.when(cmax > 0)
                def _():
                    def round_body(r, c2):
                        apply_round(dstv, msgs, cnt == r)
                        return c2

                    lax.fori_loop(1, cmax + 1, round_body, 0)

                return nxt

            lax.fori_loop(0, nbatch, batch, fetch(b, 0))

        start(0, 0)
        start(1, min(1, nchunk - 1))

        def chunk_iter(j, carry):
            for b in range(2):
                ci = j * 2 + b
                wait(b)
                process(b)

                @pl.when(ci + 2 < nchunk)
                def _():
                    start(b, ci + 2)

            return carry

        lax.fori_loop(0, nchunk // 2, chunk_iter, 0)

        zero16 = jnp.zeros((LANES,), jnp.float32)

        def finalize(i, carry):
            for k in range(FPT):
                v = abufs[k][pl.ds(i * LANES, LANES)]
                abufs[k][pl.ds(i * LANES, LANES)] = jnp.where(
                    v == -jnp.inf, zero16, v)
            return carry

        lax.fori_loop(0, ncnt, finalize, 0)

        for k in range(FPT):
            pltpu.sync_copy(abufs[k], aggT.at[fbase + k])

    return segmax


def kernel(x, edge_index, diameter, enc_W, enc_b, proc_W, proc_b, dec_W, dec_b):
    N, F = x.shape
    H = enc_W.shape[0]
    P, C = dec_b.shape
    E = edge_index.shape[1]

    NPAD = _round_up(N, LANES)
    E_pad = _round_up(E, 2 * CH)
    if NPAD == N and E_pad != E:
        NPAD += LANES
    x_p = jnp.pad(x, ((0, NPAD - N), (0, 0))) if NPAD != N else x
    if E_pad != E:
        pad = jnp.concatenate(
            [jnp.zeros((1, E_pad - E), jnp.int32),
             jnp.full((1, E_pad - E), N, jnp.int32)], axis=0)
        edges = jnp.concatenate([edge_index, pad], axis=1)
    else:
        edges = edge_index

    encoder = _make_encoder(NPAD, F, H)
    mlp = _make_mlp(NPAD, H)
    segmax = _make_segmax(NPAD, H, E_pad)
    decoder = _make_decoder(NPAD, H, C)

    hT = encoder(x_p, enc_W, enc_b[:, None])

    def body(_, hT):
        aggT = segmax(hT, edges)
        return mlp(hT, aggT, proc_W, proc_b[:, None])

    hT = lax.fori_loop(0, diameter, body, hT)

    o0, o1 = decoder(hT, dec_W[0], dec_b[0][:, None], dec_W[1], dec_b[1][:, None])
    return (o0.T[:N], o1.T[:N])


# speedup vs baseline: 3.1161x; 1.4080x over previous
import functools

import jax
import jax.numpy as jnp
from jax import lax
from jax.experimental import pallas as pl
from jax.experimental.pallas import tpu as pltpu
from jax.experimental.pallas import tpu_sc as plsc

NSUB = 32
LANES = 16
CH = 3200


def _round_up(v, m):
    return (v + m - 1) // m * m


def _make_encoder(NPAD, F, H):
    def body(x_ref, w_ref, b_ref, out_ref):
        xb = x_ref[...]
        w = w_ref[...]
        out_ref[...] = (
            lax.dot_general(w, xb, (((1,), (1,)), ((), ())),
                            preferred_element_type=jnp.float32)
            + b_ref[...]
        )

    return pl.pallas_call(
        body,
        out_shape=jax.ShapeDtypeStruct((H, NPAD), jnp.float32),
    )


def _make_mlp(NPAD, H):
    def body(h_ref, agg_ref, w_ref, b_ref, out_ref):
        s = h_ref[...] + agg_ref[...]
        out_ref[...] = (
            lax.dot_general(w_ref[...], s, (((1,), (0,)), ((), ())),
                            preferred_element_type=jnp.float32)
            + b_ref[...]
        )

    return pl.pallas_call(
        body,
        out_shape=jax.ShapeDtypeStruct((H, NPAD), jnp.float32),
    )


def _make_decoder(NPAD, H, C):
    def body(h_ref, w0_ref, b0_ref, w1_ref, b1_ref, o0_ref, o1_ref):
        h = h_ref[...]
        for w_ref, b_ref, o_ref in ((w0_ref, b0_ref, o0_ref),
                                    (w1_ref, b1_ref, o1_ref)):
            logits = lax.dot_general(
                w_ref[...], h, (((1,), (0,)), ((), ())),
                preferred_element_type=jnp.float32) + b_ref[...]
            m = jnp.max(logits, axis=0, keepdims=True)
            e = jnp.exp(logits - m)
            s = jnp.sum(e, axis=0, keepdims=True)
            o_ref[...] = logits - m - jnp.log(s)

    return pl.pallas_call(
        body,
        out_shape=(jax.ShapeDtypeStruct((C, NPAD), jnp.float32),
                   jax.ShapeDtypeStruct((C, NPAD), jnp.float32)),
    )


def _make_segmax(NPAD, H, E_pad):
    FPT = H // NSUB
    nchunk = E_pad // CH
    nbatch = CH // LANES
    ncnt = NPAD // LANES
    ndirty = _round_up(nbatch, LANES) + LANES

    mesh = plsc.VectorSubcoreMesh(core_axis_name="c", subcore_axis_name="s")

    @functools.partial(
        pl.kernel,
        out_type=jax.ShapeDtypeStruct((H, NPAD), jnp.float32),
        mesh=mesh,
        compiler_params=pltpu.CompilerParams(
            needs_layout_passes=False, use_tc_tiling_on_sc=False),
        scratch_types=[
            [pltpu.VMEM((NPAD,), jnp.float32) for _ in range(FPT)],
            [pltpu.VMEM((NPAD,), jnp.float32) for _ in range(FPT)],
            pltpu.VMEM((2, 2, CH), jnp.int32),
            pltpu.VMEM((NPAD,), jnp.int32),
            pltpu.VMEM((ndirty,), jnp.int32),
            pltpu.SemaphoreType.DMA,
            pltpu.SemaphoreType.DMA,
        ],
    )
    def segmax(hT, edges, aggT, hbufs, abufs, eb, tmp, dirty, semA, semB):
        c = lax.axis_index("c")
        s = lax.axis_index("s")
        wid = s * 2 + c
        fbase = wid * FPT
        sems = (semA, semB)

        for k in range(FPT):
            pltpu.make_async_copy(hT.at[fbase + k], hbufs[k], semA).start()

        neg = jnp.full((LANES,), -jnp.inf, jnp.float32)

        def init_agg(i, carry):
            for k in range(FPT):
                abufs[k][pl.ds(i * LANES, LANES)] = neg
            return carry

        lax.fori_loop(0, ncnt, init_agg, 0)

        zeros16i = jnp.zeros((LANES,), jnp.int32)

        def init_dirty(i, carry):
            dirty[pl.ds(i * LANES, LANES)] = zeros16i
            return carry

        lax.fori_loop(0, ndirty // LANES, init_dirty, 0)

        for k in range(FPT):
            pltpu.make_async_copy(hT.at[fbase + k], hbufs[k], semA).wait()

        def start(b, ci):
            off = ci * CH
            pltpu.make_async_copy(edges.at[0, pl.ds(off, CH)],
                                  eb.at[b, 0], sems[b]).start()
            pltpu.make_async_copy(edges.at[1, pl.ds(off, CH)],
                                  eb.at[b, 1], sems[b]).start()

        def wait(b):
            pltpu.make_async_copy(edges.at[0, pl.ds(0, CH)],
                                  eb.at[b, 0], sems[b]).wait()
            pltpu.make_async_copy(edges.at[1, pl.ds(0, CH)],
                                  eb.at[b, 1], sems[b]).wait()

        lane = lax.iota(jnp.int32, LANES)
        lane0 = lane == 0

        def apply_round(dstv, msgs, w):
            curs = [plsc.load_gather(abufs[k], [dstv]) for k in range(FPT)]
            news = [jnp.maximum(curs[k], msgs[k]) for k in range(FPT)]
            for k in range(FPT):
                plsc.store_scatter(abufs[k], [dstv], news[k], mask=w)

        def load_batch(b, base):
            srcv = eb[b, 0, pl.ds(base, LANES)]
            dstv = eb[b, 1, pl.ds(base, LANES)]
            msgs = [plsc.load_gather(hbufs[k], [srcv]) for k in range(FPT)]
            return dstv, msgs

        def process(b):
            def batch(t, carry):
                dstv, msgs = load_batch(b, t * LANES)
                plsc.store_scatter(tmp, [dstv], lane)
                got = plsc.load_gather(tmp, [dstv])
                w = got == lane
                apply_round(dstv, msgs, w)
                nlost = plsc.all_reduce_population_count(jnp.logical_not(w))
                tvec = jnp.zeros((LANES,), jnp.int32) + t
                plsc.store_scatter(dirty, [tvec], nlost, mask=lane0)
                return carry

            lax.fori_loop(0, nbatch, batch, 0)

            def rerun(t):
                dstv, msgs = load_batch(b, t * LANES)
                cnt, _ = plsc.scan_count(dstv)
                cmax = jnp.max(cnt)

                def round_body(r, c2):
                    apply_round(dstv, msgs, cnt == r)
                    return c2

                lax.fori_loop(0, cmax + 1, round_body, 0)

            def scan_group(g, carry):
                dvec = dirty[pl.ds(g * LANES, LANES)]

                @pl.when(jnp.any(dvec != 0))
                def _():
                    def inner(i, c2):
                        t = g * LANES + i
                        dv = dirty[pl.ds(t, LANES)]

                        @pl.when(dv[0] != 0)
                        def _():
                            rerun(t)

                        return c2

                    lax.fori_loop(0, LANES, inner, 0)

                return carry

            lax.fori_loop(0, (ndirty - LANES) // LANES, scan_group, 0)

        start(0, 0)
        start(1, min(1, nchunk - 1))

        def chunk_iter(j, carry):
            for b in range(2):
                ci = j * 2 + b
                wait(b)
                process(b)

                @pl.when(ci + 2 < nchunk)
                def _():
                    start(b, ci + 2)

            return carry

        lax.fori_loop(0, nchunk // 2, chunk_iter, 0)

        zero16 = jnp.zeros((LANES,), jnp.float32)

        def finalize(i, carry):
            for k in range(FPT):
                v = abufs[k][pl.ds(i * LANES, LANES)]
                abufs[k][pl.ds(i * LANES, LANES)] = jnp.where(
                    v == -jnp.inf, zero16, v)
            return carry

        lax.fori_loop(0, ncnt, finalize, 0)

        for k in range(FPT):
            pltpu.sync_copy(abufs[k], aggT.at[fbase + k])

    return segmax


def kernel(x, edge_index, diameter, enc_W, enc_b, proc_W, proc_b, dec_W, dec_b):
    N, F = x.shape
    H = enc_W.shape[0]
    P, C = dec_b.shape
    E = edge_index.shape[1]

    NPAD = _round_up(N, LANES)
    E_pad = _round_up(E, 2 * CH)
    if NPAD == N and E_pad != E:
        NPAD += LANES
    x_p = jnp.pad(x, ((0, NPAD - N), (0, 0))) if NPAD != N else x
    if E_pad != E:
        pad = jnp.concatenate(
            [jnp.zeros((1, E_pad - E), jnp.int32),
             jnp.full((1, E_pad - E), N, jnp.int32)], axis=0)
        edges = jnp.concatenate([edge_index, pad], axis=1)
    else:
        edges = edge_index

    encoder = _make_encoder(NPAD, F, H)
    mlp = _make_mlp(NPAD, H)
    segmax = _make_segmax(NPAD, H, E_pad)
    decoder = _make_decoder(NPAD, H, C)

    hT = encoder(x_p, enc_W, enc_b[:, None])

    def body(_, hT):
        aggT = segmax(hT, edges)
        return mlp(hT, aggT, proc_W, proc_b[:, None])

    hT = lax.fori_loop(0, diameter, body, hT)

    o0, o1 = decoder(hT, dec_W[0], dec_b[0][:, None], dec_W[1], dec_b[1][:, None])
    return (o0.T[:N], o1.T[:N])


# speedup vs baseline: 3.1216x; 1.0018x over previous
import functools

import jax
import jax.numpy as jnp
from jax import lax
from jax.experimental import pallas as pl
from jax.experimental.pallas import tpu as pltpu
from jax.experimental.pallas import tpu_sc as plsc

NSUB = 32
LANES = 16
CH = 3200


def _round_up(v, m):
    return (v + m - 1) // m * m


def _make_encoder(NPAD, F, H):
    def body(x_ref, w_ref, b_ref, out_ref):
        xb = x_ref[...]
        w = w_ref[...]
        out_ref[...] = (
            lax.dot_general(w, xb, (((1,), (1,)), ((), ())),
                            preferred_element_type=jnp.float32)
            + b_ref[...]
        )

    return pl.pallas_call(
        body,
        out_shape=jax.ShapeDtypeStruct((H, NPAD), jnp.float32),
    )


def _make_mlp(NPAD, H):
    def body(h_ref, agg_ref, w_ref, b_ref, out_ref):
        s = h_ref[...] + agg_ref[...]
        out_ref[...] = (
            lax.dot_general(w_ref[...], s, (((1,), (0,)), ((), ())),
                            preferred_element_type=jnp.float32)
            + b_ref[...]
        )

    return pl.pallas_call(
        body,
        out_shape=jax.ShapeDtypeStruct((H, NPAD), jnp.float32),
    )


def _make_decoder(NPAD, H, C):
    def body(h_ref, w0_ref, b0_ref, w1_ref, b1_ref, o0_ref, o1_ref):
        h = h_ref[...]
        for w_ref, b_ref, o_ref in ((w0_ref, b0_ref, o0_ref),
                                    (w1_ref, b1_ref, o1_ref)):
            logits = lax.dot_general(
                w_ref[...], h, (((1,), (0,)), ((), ())),
                preferred_element_type=jnp.float32) + b_ref[...]
            m = jnp.max(logits, axis=0, keepdims=True)
            e = jnp.exp(logits - m)
            s = jnp.sum(e, axis=0, keepdims=True)
            o_ref[...] = logits - m - jnp.log(s)

    return pl.pallas_call(
        body,
        out_shape=(jax.ShapeDtypeStruct((C, NPAD), jnp.float32),
                   jax.ShapeDtypeStruct((C, NPAD), jnp.float32)),
    )


def _make_segmax(NPAD, H, E_pad):
    FPT = H // NSUB
    nchunk = E_pad // CH
    nbatch = CH // LANES
    ncnt = NPAD // LANES
    ndirty = _round_up(nbatch, LANES) + LANES

    mesh = plsc.VectorSubcoreMesh(core_axis_name="c", subcore_axis_name="s")

    @functools.partial(
        pl.kernel,
        out_type=jax.ShapeDtypeStruct((H, NPAD), jnp.float32),
        mesh=mesh,
        compiler_params=pltpu.CompilerParams(
            needs_layout_passes=False, use_tc_tiling_on_sc=False),
        scratch_types=[
            [pltpu.VMEM((NPAD,), jnp.float32) for _ in range(FPT)],
            [pltpu.VMEM((NPAD,), jnp.float32) for _ in range(FPT)],
            pltpu.VMEM((2, 2, CH), jnp.int32),
            [pltpu.VMEM((NPAD,), jnp.int32) for _ in range(2)],
            pltpu.VMEM((ndirty,), jnp.int32),
            pltpu.SemaphoreType.DMA,
            pltpu.SemaphoreType.DMA,
        ],
    )
    def segmax(hT, edges, aggT, hbufs, abufs, eb, tmps, dirty, semA, semB):
        c = lax.axis_index("c")
        s = lax.axis_index("s")
        wid = s * 2 + c
        fbase = wid * FPT
        sems = (semA, semB)

        for k in range(FPT):
            pltpu.make_async_copy(hT.at[fbase + k], hbufs[k], semA).start()

        neg = jnp.full((LANES,), -jnp.inf, jnp.float32)

        def init_agg(i, carry):
            for k in range(FPT):
                abufs[k][pl.ds(i * LANES, LANES)] = neg
            return carry

        lax.fori_loop(0, ncnt, init_agg, 0)

        zeros16i = jnp.zeros((LANES,), jnp.int32)

        def init_dirty(i, carry):
            dirty[pl.ds(i * LANES, LANES)] = zeros16i
            return carry

        lax.fori_loop(0, ndirty // LANES, init_dirty, 0)

        for k in range(FPT):
            pltpu.make_async_copy(hT.at[fbase + k], hbufs[k], semA).wait()

        def start(b, ci):
            off = ci * CH
            pltpu.make_async_copy(edges.at[0, pl.ds(off, CH)],
                                  eb.at[b, 0], sems[b]).start()
            pltpu.make_async_copy(edges.at[1, pl.ds(off, CH)],
                                  eb.at[b, 1], sems[b]).start()

        def wait(b):
            pltpu.make_async_copy(edges.at[0, pl.ds(0, CH)],
                                  eb.at[b, 0], sems[b]).wait()
            pltpu.make_async_copy(edges.at[1, pl.ds(0, CH)],
                                  eb.at[b, 1], sems[b]).wait()

        lane = lax.iota(jnp.int32, LANES)
        lane0 = lane == 0

        def apply_round(dstv, msgs, w):
            curs = [plsc.load_gather(abufs[k], [dstv]) for k in range(FPT)]
            news = [jnp.maximum(curs[k], msgs[k]) for k in range(FPT)]
            for k in range(FPT):
                plsc.store_scatter(abufs[k], [dstv], news[k], mask=w)

        def load_batch(b, base):
            srcv = eb[b, 0, pl.ds(base, LANES)]
            dstv = eb[b, 1, pl.ds(base, LANES)]
            msgs = [plsc.load_gather(hbufs[k], [srcv]) for k in range(FPT)]
            return dstv, msgs

        def process(b):
            def one_batch(t, tmp):
                dstv, msgs = load_batch(b, t * LANES)
                plsc.store_scatter(tmp, [dstv], lane)
                got = plsc.load_gather(tmp, [dstv])
                w = got == lane
                apply_round(dstv, msgs, w)
                nlost = plsc.all_reduce_population_count(jnp.logical_not(w))
                tvec = jnp.zeros((LANES,), jnp.int32) + t
                plsc.store_scatter(dirty, [tvec], nlost, mask=lane0)

            def batch2(u, carry):
                one_batch(u * 2, tmps[0])
                one_batch(u * 2 + 1, tmps[1])
                return carry

            lax.fori_loop(0, nbatch // 2, batch2, 0)

            def rerun(t):
                dstv, msgs = load_batch(b, t * LANES)
                cnt, _ = plsc.scan_count(dstv)
                cmax = jnp.max(cnt)

                def round_body(r, c2):
                    apply_round(dstv, msgs, cnt == r)
                    return c2

                lax.fori_loop(0, cmax + 1, round_body, 0)

            def scan_group(g, carry):
                dvec = dirty[pl.ds(g * LANES, LANES)]

                @pl.when(jnp.any(dvec != 0))
                def _():
                    def inner(i, c2):
                        t = g * LANES + i
                        dv = dirty[pl.ds(t, LANES)]

                        @pl.when(dv[0] != 0)
                        def _():
                            rerun(t)

                        return c2

                    lax.fori_loop(0, LANES, inner, 0)

                return carry

            lax.fori_loop(0, (ndirty - LANES) // LANES, scan_group, 0)

        start(0, 0)
        start(1, min(1, nchunk - 1))

        def chunk_iter(j, carry):
            for b in range(2):
                ci = j * 2 + b
                wait(b)
                process(b)

                @pl.when(ci + 2 < nchunk)
                def _():
                    start(b, ci + 2)

            return carry

        lax.fori_loop(0, nchunk // 2, chunk_iter, 0)

        zero16 = jnp.zeros((LANES,), jnp.float32)

        def finalize(i, carry):
            for k in range(FPT):
                v = abufs[k][pl.ds(i * LANES, LANES)]
                abufs[k][pl.ds(i * LANES, LANES)] = jnp.where(
                    v == -jnp.inf, zero16, v)
            return carry

        lax.fori_loop(0, ncnt, finalize, 0)

        for k in range(FPT):
            pltpu.sync_copy(abufs[k], aggT.at[fbase + k])

    return segmax


def kernel(x, edge_index, diameter, enc_W, enc_b, proc_W, proc_b, dec_W, dec_b):
    N, F = x.shape
    H = enc_W.shape[0]
    P, C = dec_b.shape
    E = edge_index.shape[1]

    NPAD = _round_up(N, LANES)
    E_pad = _round_up(E, 2 * CH)
    if NPAD == N and E_pad != E:
        NPAD += LANES
    x_p = jnp.pad(x, ((0, NPAD - N), (0, 0))) if NPAD != N else x
    if E_pad != E:
        pad = jnp.concatenate(
            [jnp.zeros((1, E_pad - E), jnp.int32),
             jnp.full((1, E_pad - E), N, jnp.int32)], axis=0)
        edges = jnp.concatenate([edge_index, pad], axis=1)
    else:
        edges = edge_index

    encoder = _make_encoder(NPAD, F, H)
    mlp = _make_mlp(NPAD, H)
    segmax = _make_segmax(NPAD, H, E_pad)
    decoder = _make_decoder(NPAD, H, C)

    hT = encoder(x_p, enc_W, enc_b[:, None])

    def body(_, hT):
        aggT = segmax(hT, edges)
        return mlp(hT, aggT, proc_W, proc_b[:, None])

    hT = lax.fori_loop(0, diameter, body, hT)

    o0, o1 = decoder(hT, dec_W[0], dec_b[0][:, None], dec_W[1], dec_b[1][:, None])
    return (o0.T[:N], o1.T[:N])


# speedup vs baseline: 3.7839x; 1.2122x over previous
import functools

import jax
import jax.numpy as jnp
from jax import lax
from jax.experimental import pallas as pl
from jax.experimental.pallas import tpu as pltpu
from jax.experimental.pallas import tpu_sc as plsc

NSUB = 32
LANES = 16
CH = 3200


def _round_up(v, m):
    return (v + m - 1) // m * m


def _make_encoder(NPAD, F, H):
    def body(x_ref, w_ref, b_ref, out_ref):
        xb = x_ref[...]
        w = w_ref[...]
        out_ref[...] = (
            lax.dot_general(w, xb, (((1,), (1,)), ((), ())),
                            preferred_element_type=jnp.float32)
            + b_ref[...]
        )

    return pl.pallas_call(
        body,
        out_shape=jax.ShapeDtypeStruct((H, NPAD), jnp.float32),
    )


def _make_mlp(NPAD, H):
    def body(h_ref, agg_ref, w_ref, b_ref, out_ref):
        s = h_ref[...] + agg_ref[...]
        out_ref[...] = (
            lax.dot_general(w_ref[...], s, (((1,), (0,)), ((), ())),
                            preferred_element_type=jnp.float32)
            + b_ref[...]
        )

    return pl.pallas_call(
        body,
        out_shape=jax.ShapeDtypeStruct((H, NPAD), jnp.float32),
    )


def _make_decoder(NPAD, H, C):
    def body(h_ref, w0_ref, b0_ref, w1_ref, b1_ref, o0_ref, o1_ref):
        h = h_ref[...]
        for w_ref, b_ref, o_ref in ((w0_ref, b0_ref, o0_ref),
                                    (w1_ref, b1_ref, o1_ref)):
            logits = lax.dot_general(
                w_ref[...], h, (((1,), (0,)), ((), ())),
                preferred_element_type=jnp.float32) + b_ref[...]
            m = jnp.max(logits, axis=0, keepdims=True)
            e = jnp.exp(logits - m)
            s = jnp.sum(e, axis=0, keepdims=True)
            o_ref[...] = logits - m - jnp.log(s)

    return pl.pallas_call(
        body,
        out_shape=(jax.ShapeDtypeStruct((C, NPAD), jnp.float32),
                   jax.ShapeDtypeStruct((C, NPAD), jnp.float32)),
    )


def _make_segmax(NPAD, H, E_pad):
    FPT = H // NSUB
    nchunk = E_pad // CH
    nbatch = CH // LANES
    ncnt = NPAD // LANES
    ndirty = _round_up(nbatch, LANES) + LANES

    mesh = plsc.VectorSubcoreMesh(core_axis_name="c", subcore_axis_name="s")

    @functools.partial(
        pl.kernel,
        out_type=jax.ShapeDtypeStruct((H, NPAD), jnp.float32),
        mesh=mesh,
        compiler_params=pltpu.CompilerParams(
            needs_layout_passes=False, use_tc_tiling_on_sc=False),
        scratch_types=[
            [pltpu.VMEM((NPAD,), jnp.float32) for _ in range(FPT)],
            [pltpu.VMEM((NPAD,), jnp.float32) for _ in range(FPT)],
            pltpu.VMEM((2, 2, CH), jnp.int32),
            [pltpu.VMEM((NPAD,), jnp.int32) for _ in range(2)],
            pltpu.VMEM((ndirty,), jnp.int32),
            pltpu.SemaphoreType.DMA,
            pltpu.SemaphoreType.DMA,
        ],
    )
    def segmax(hT, edges, aggT, hbufs, abufs, eb, tmps, dirty, semA, semB):
        c = lax.axis_index("c")
        s = lax.axis_index("s")
        wid = s * 2 + c
        fbase = wid * FPT
        sems = (semA, semB)

        for k in range(FPT):
            pltpu.make_async_copy(hT.at[fbase + k], hbufs[k], semA).start()

        neg = jnp.full((LANES,), -jnp.inf, jnp.float32)

        def init_agg(i, carry):
            for k in range(FPT):
                abufs[k][pl.ds(i * LANES, LANES)] = neg
            return carry

        lax.fori_loop(0, ncnt, init_agg, 0)

        zeros16i = jnp.zeros((LANES,), jnp.int32)

        def init_dirty(i, carry):
            dirty[pl.ds(i * LANES, LANES)] = zeros16i
            return carry

        lax.fori_loop(0, ndirty // LANES, init_dirty, 0)

        for k in range(FPT):
            pltpu.make_async_copy(hT.at[fbase + k], hbufs[k], semA).wait()

        def start(b, ci):
            off = ci * CH
            pltpu.make_async_copy(edges.at[0, pl.ds(off, CH)],
                                  eb.at[b, 0], sems[b]).start()
            pltpu.make_async_copy(edges.at[1, pl.ds(off, CH)],
                                  eb.at[b, 1], sems[b]).start()

        def wait(b):
            pltpu.make_async_copy(edges.at[0, pl.ds(0, CH)],
                                  eb.at[b, 0], sems[b]).wait()
            pltpu.make_async_copy(edges.at[1, pl.ds(0, CH)],
                                  eb.at[b, 1], sems[b]).wait()

        lane = lax.iota(jnp.int32, LANES)
        lane0 = lane == 0

        def apply_round(dstv, msgs, w):
            curs = [plsc.load_gather(abufs[k], [dstv]) for k in range(FPT)]
            news = [jnp.maximum(curs[k], msgs[k]) for k in range(FPT)]
            for k in range(FPT):
                plsc.store_scatter(abufs[k], [dstv], news[k], mask=w)

        def load_batch(b, base):
            srcv = eb[b, 0, pl.ds(base, LANES)]
            dstv = eb[b, 1, pl.ds(base, LANES)]
            msgs = [plsc.load_gather(hbufs[k], [srcv]) for k in range(FPT)]
            return dstv, msgs

        npair = nbatch // 2
        zl = jnp.zeros((LANES,), jnp.int32)

        def fetch_pair(b, u):
            out = []
            for half in range(2):
                t = u * 2 + half
                dstv, msgs = load_batch(b, t * LANES)
                plsc.store_scatter(tmps[half], [dstv], lane)
                got = plsc.load_gather(tmps[half], [dstv])
                w = got == lane
                nlost = plsc.all_reduce_population_count(jnp.logical_not(w))
                plsc.store_scatter(dirty, [zl + t], nlost, mask=lane0)
                out += [dstv, w, *msgs]
            return tuple(out)

        def process(b):
            def pair_body(u, carry):
                (dstA, wA, mA0, mA1, mA2, mA3,
                 dstB, wB, mB0, mB1, mB2, mB3) = carry
                un = jnp.minimum(u + 1, npair - 1)
                tA = un * 2
                tB = tA + 1
                sA = eb[b, 0, pl.ds(tA * LANES, LANES)]
                dA = eb[b, 1, pl.ds(tA * LANES, LANES)]
                sB = eb[b, 0, pl.ds(tB * LANES, LANES)]
                dB = eb[b, 1, pl.ds(tB * LANES, LANES)]
                plsc.store_scatter(tmps[0], [dA], lane)
                plsc.store_scatter(tmps[1], [dB], lane)
                nA = [plsc.load_gather(hbufs[k], [sA]) for k in range(FPT)]
                nB = [plsc.load_gather(hbufs[k], [sB]) for k in range(FPT)]
                gA = plsc.load_gather(tmps[0], [dA])
                gB = plsc.load_gather(tmps[1], [dB])
                apply_round(dstA, [mA0, mA1, mA2, mA3], wA)
                nwA = gA == lane
                nwB = gB == lane
                nlA = plsc.all_reduce_population_count(jnp.logical_not(nwA))
                nlB = plsc.all_reduce_population_count(jnp.logical_not(nwB))
                plsc.store_scatter(dirty, [zl + tA], nlA, mask=lane0)
                plsc.store_scatter(dirty, [zl + tB], nlB, mask=lane0)
                apply_round(dstB, [mB0, mB1, mB2, mB3], wB)
                return (dA, nwA, *nA, dB, nwB, *nB)

            lax.fori_loop(0, npair, pair_body, fetch_pair(b, 0))

            def rerun(t):
                dstv, msgs = load_batch(b, t * LANES)
                cnt, _ = plsc.scan_count(dstv)
                cmax = jnp.max(cnt)

                def round_body(r, c2):
                    apply_round(dstv, msgs, cnt == r)
                    return c2

                lax.fori_loop(0, cmax + 1, round_body, 0)

            def scan_group(g, carry):
                dvec = dirty[pl.ds(g * LANES, LANES)]

                @pl.when(jnp.any(dvec != 0))
                def _():
                    def inner(i, c2):
                        t = g * LANES + i
                        dv = dirty[pl.ds(t, LANES)]

                        @pl.when(dv[0] != 0)
                        def _():
                            rerun(t)

                        return c2

                    lax.fori_loop(0, LANES, inner, 0)

                return carry

            lax.fori_loop(0, (ndirty - LANES) // LANES, scan_group, 0)

        start(0, 0)
        start(1, min(1, nchunk - 1))

        def chunk_iter(j, carry):
            for b in range(2):
                ci = j * 2 + b
                wait(b)
                process(b)

                @pl.when(ci + 2 < nchunk)
                def _():
                    start(b, ci + 2)

            return carry

        lax.fori_loop(0, nchunk // 2, chunk_iter, 0)

        zero16 = jnp.zeros((LANES,), jnp.float32)

        def finalize(i, carry):
            for k in range(FPT):
                v = abufs[k][pl.ds(i * LANES, LANES)]
                abufs[k][pl.ds(i * LANES, LANES)] = jnp.where(
                    v == -jnp.inf, zero16, v)
            return carry

        lax.fori_loop(0, ncnt, finalize, 0)

        for k in range(FPT):
            pltpu.sync_copy(abufs[k], aggT.at[fbase + k])

    return segmax


def kernel(x, edge_index, diameter, enc_W, enc_b, proc_W, proc_b, dec_W, dec_b):
    N, F = x.shape
    H = enc_W.shape[0]
    P, C = dec_b.shape
    E = edge_index.shape[1]

    NPAD = _round_up(N, LANES)
    E_pad = _round_up(E, 2 * CH)
    if NPAD == N and E_pad != E:
        NPAD += LANES
    x_p = jnp.pad(x, ((0, NPAD - N), (0, 0))) if NPAD != N else x
    if E_pad != E:
        pad = jnp.concatenate(
            [jnp.zeros((1, E_pad - E), jnp.int32),
             jnp.full((1, E_pad - E), N, jnp.int32)], axis=0)
        edges = jnp.concatenate([edge_index, pad], axis=1)
    else:
        edges = edge_index

    encoder = _make_encoder(NPAD, F, H)
    mlp = _make_mlp(NPAD, H)
    segmax = _make_segmax(NPAD, H, E_pad)
    decoder = _make_decoder(NPAD, H, C)

    hT = encoder(x_p, enc_W, enc_b[:, None])

    def body(_, hT):
        aggT = segmax(hT, edges)
        return mlp(hT, aggT, proc_W, proc_b[:, None])

    hT = lax.fori_loop(0, diameter, body, hT)

    o0, o1 = decoder(hT, dec_W[0], dec_b[0][:, None], dec_W[1], dec_b[1][:, None])
    return (o0.T[:N], o1.T[:N])


# speedup vs baseline: 4.3335x; 1.1452x over previous
import functools

import jax
import jax.numpy as jnp
from jax import lax
from jax.experimental import pallas as pl
from jax.experimental.pallas import tpu as pltpu
from jax.experimental.pallas import tpu_sc as plsc

NSUB = 32
LANES = 16
CH = 3200


def _round_up(v, m):
    return (v + m - 1) // m * m


def _make_encoder(NPAD, F, H):
    def body(x_ref, w_ref, b_ref, out_ref):
        xb = x_ref[...]
        w = w_ref[...]
        out_ref[...] = (
            lax.dot_general(w, xb, (((1,), (1,)), ((), ())),
                            preferred_element_type=jnp.float32)
            + b_ref[...]
        )

    return pl.pallas_call(
        body,
        out_shape=jax.ShapeDtypeStruct((H, NPAD), jnp.float32),
    )


def _make_mlp(NPAD, H):
    def body(h_ref, agg_ref, w_ref, b_ref, out_ref):
        s = h_ref[...] + agg_ref[...]
        out_ref[...] = (
            lax.dot_general(w_ref[...], s, (((1,), (0,)), ((), ())),
                            preferred_element_type=jnp.float32)
            + b_ref[...]
        )

    return pl.pallas_call(
        body,
        out_shape=jax.ShapeDtypeStruct((H, NPAD), jnp.float32),
    )


def _make_decoder(NPAD, H, C):
    def body(h_ref, w0_ref, b0_ref, w1_ref, b1_ref, o0_ref, o1_ref):
        h = h_ref[...]
        for w_ref, b_ref, o_ref in ((w0_ref, b0_ref, o0_ref),
                                    (w1_ref, b1_ref, o1_ref)):
            logits = lax.dot_general(
                w_ref[...], h, (((1,), (0,)), ((), ())),
                preferred_element_type=jnp.float32) + b_ref[...]
            m = jnp.max(logits, axis=0, keepdims=True)
            e = jnp.exp(logits - m)
            s = jnp.sum(e, axis=0, keepdims=True)
            o_ref[...] = logits - m - jnp.log(s)

    return pl.pallas_call(
        body,
        out_shape=(jax.ShapeDtypeStruct((C, NPAD), jnp.float32),
                   jax.ShapeDtypeStruct((C, NPAD), jnp.float32)),
    )


def _make_segmax(NPAD, H, E_pad):
    FPT = H // NSUB
    PPT = FPT // 2
    nchunk = E_pad // CH
    nbatch = CH // LANES
    ncnt = NPAD // LANES
    ndirty = _round_up(nbatch, LANES) + LANES

    mesh = plsc.VectorSubcoreMesh(core_axis_name="c", subcore_axis_name="s")

    @functools.partial(
        pl.kernel,
        out_type=jax.ShapeDtypeStruct((H, NPAD), jnp.float32),
        mesh=mesh,
        compiler_params=pltpu.CompilerParams(
            needs_layout_passes=False, use_tc_tiling_on_sc=False),
        scratch_types=[
            [pltpu.VMEM((NPAD,), jnp.float32) for _ in range(FPT)],
            [pltpu.VMEM((NPAD,), jnp.int32) for _ in range(PPT)],
            [pltpu.VMEM((NPAD,), jnp.int32) for _ in range(PPT)],
            pltpu.VMEM((2, 2, CH), jnp.int32),
            [pltpu.VMEM((NPAD,), jnp.int32) for _ in range(2)],
            pltpu.VMEM((ndirty,), jnp.int32),
            pltpu.SemaphoreType.DMA,
            pltpu.SemaphoreType.DMA,
        ],
    )
    def segmax(hT, edges, aggT, fbufs, hbufs, abufs, eb, tmps, dirty,
               semA, semB):
        c = lax.axis_index("c")
        s = lax.axis_index("s")
        wid = s * 2 + c
        fbase = wid * FPT
        sems = (semA, semB)

        for k in range(FPT):
            pltpu.make_async_copy(hT.at[fbase + k], fbufs[k], semA).start()

        zeros16i = jnp.zeros((LANES,), jnp.int32)

        def init_dirty(i, carry):
            dirty[pl.ds(i * LANES, LANES)] = zeros16i
            return carry

        lax.fori_loop(0, ndirty // LANES, init_dirty, 0)

        for k in range(FPT):
            pltpu.make_async_copy(hT.at[fbase + k], fbufs[k], semA).wait()

        ninf = jnp.full((LANES,), -jnp.inf, jnp.float32)
        neg2 = plsc.bitcast(
            plsc.pack(ninf, ninf, format=plsc.PackFormat.INTERLEAVED),
            jnp.int32)

        def pack_init(i, carry):
            sl = pl.ds(i * LANES, LANES)
            for p in range(PPT):
                packed = plsc.pack(fbufs[2 * p][sl], fbufs[2 * p + 1][sl],
                                   format=plsc.PackFormat.INTERLEAVED)
                hbufs[p][sl] = plsc.bitcast(packed, jnp.int32)
                abufs[p][sl] = neg2
            return carry

        lax.fori_loop(0, ncnt, pack_init, 0)

        def start(b, ci):
            off = ci * CH
            pltpu.make_async_copy(edges.at[0, pl.ds(off, CH)],
                                  eb.at[b, 0], sems[b]).start()
            pltpu.make_async_copy(edges.at[1, pl.ds(off, CH)],
                                  eb.at[b, 1], sems[b]).start()

        def wait(b):
            pltpu.make_async_copy(edges.at[0, pl.ds(0, CH)],
                                  eb.at[b, 0], sems[b]).wait()
            pltpu.make_async_copy(edges.at[1, pl.ds(0, CH)],
                                  eb.at[b, 1], sems[b]).wait()

        lane = lax.iota(jnp.int32, LANES)
        lane0 = lane == 0

        def bmax(a_i32, b_i32):
            a = plsc.bitcast(a_i32, jnp.bfloat16)
            b = plsc.bitcast(b_i32, jnp.bfloat16)
            return plsc.bitcast(jnp.maximum(a, b), jnp.int32)

        def apply_round(dstv, msgs, w):
            curs = [plsc.load_gather(abufs[p], [dstv]) for p in range(PPT)]
            news = [bmax(curs[p], msgs[p]) for p in range(PPT)]
            for p in range(PPT):
                plsc.store_scatter(abufs[p], [dstv], news[p], mask=w)

        def load_batch(b, base):
            srcv = eb[b, 0, pl.ds(base, LANES)]
            dstv = eb[b, 1, pl.ds(base, LANES)]
            msgs = [plsc.load_gather(hbufs[p], [srcv]) for p in range(PPT)]
            return dstv, msgs

        npair = nbatch // 2
        zl = jnp.zeros((LANES,), jnp.int32)

        def fetch_pair(b, u):
            out = []
            for half in range(2):
                t = u * 2 + half
                dstv, msgs = load_batch(b, t * LANES)
                plsc.store_scatter(tmps[half], [dstv], lane)
                got = plsc.load_gather(tmps[half], [dstv])
                w = got == lane
                nlost = plsc.all_reduce_population_count(jnp.logical_not(w))
                plsc.store_scatter(dirty, [zl + t], nlost, mask=lane0)
                out += [dstv, w, *msgs]
            return tuple(out)

        def process(b):
            def pair_body(u, carry):
                (dstA, wA, mA0, mA1,
                 dstB, wB, mB0, mB1) = carry
                un = jnp.minimum(u + 1, npair - 1)
                tA = un * 2
                tB = tA + 1
                sA = eb[b, 0, pl.ds(tA * LANES, LANES)]
                dA = eb[b, 1, pl.ds(tA * LANES, LANES)]
                sB = eb[b, 0, pl.ds(tB * LANES, LANES)]
                dB = eb[b, 1, pl.ds(tB * LANES, LANES)]
                plsc.store_scatter(tmps[0], [dA], lane)
                plsc.store_scatter(tmps[1], [dB], lane)
                nA = [plsc.load_gather(hbufs[p], [sA]) for p in range(PPT)]
                nB = [plsc.load_gather(hbufs[p], [sB]) for p in range(PPT)]
                gA = plsc.load_gather(tmps[0], [dA])
                gB = plsc.load_gather(tmps[1], [dB])
                apply_round(dstA, [mA0, mA1], wA)
                nwA = gA == lane
                nwB = gB == lane
                nlA = plsc.all_reduce_population_count(jnp.logical_not(nwA))
                nlB = plsc.all_reduce_population_count(jnp.logical_not(nwB))
                plsc.store_scatter(dirty, [zl + tA], nlA, mask=lane0)
                plsc.store_scatter(dirty, [zl + tB], nlB, mask=lane0)
                apply_round(dstB, [mB0, mB1], wB)
                return (dA, nwA, *nA, dB, nwB, *nB)

            lax.fori_loop(0, npair, pair_body, fetch_pair(b, 0))

            def rerun(t):
                dstv, msgs = load_batch(b, t * LANES)
                cnt, _ = plsc.scan_count(dstv)
                cmax = jnp.max(cnt)

                def round_body(r, c2):
                    apply_round(dstv, msgs, cnt == r)
                    return c2

                lax.fori_loop(0, cmax + 1, round_body, 0)

            def scan_group(g, carry):
                dvec = dirty[pl.ds(g * LANES, LANES)]

                @pl.when(jnp.any(dvec != 0))
                def _():
                    def inner(i, c2):
                        t = g * LANES + i
                        dv = dirty[pl.ds(t, LANES)]

                        @pl.when(dv[0] != 0)
                        def _():
                            rerun(t)

                        return c2

                    lax.fori_loop(0, LANES, inner, 0)

                return carry

            lax.fori_loop(0, (ndirty - LANES) // LANES, scan_group, 0)

        start(0, 0)
        start(1, min(1, nchunk - 1))

        def chunk_iter(j, carry):
            for b in range(2):
                ci = j * 2 + b
                wait(b)
                process(b)

                @pl.when(ci + 2 < nchunk)
                def _():
                    start(b, ci + 2)

            return carry

        lax.fori_loop(0, nchunk // 2, chunk_iter, 0)

        zero16 = jnp.zeros((LANES,), jnp.float32)

        def finalize(i, carry):
            sl = pl.ds(i * LANES, LANES)
            for p in range(PPT):
                vbf = plsc.bitcast(abufs[p][sl], jnp.bfloat16)
                va, vb = plsc.unpack(vbf, format=plsc.PackFormat.INTERLEAVED)
                fbufs[2 * p][sl] = jnp.where(va == -jnp.inf, zero16, va)
                fbufs[2 * p + 1][sl] = jnp.where(vb == -jnp.inf, zero16, vb)
            return carry

        lax.fori_loop(0, ncnt, finalize, 0)

        for k in range(FPT):
            pltpu.sync_copy(fbufs[k], aggT.at[fbase + k])

    return segmax


def kernel(x, edge_index, diameter, enc_W, enc_b, proc_W, proc_b, dec_W, dec_b):
    N, F = x.shape
    H = enc_W.shape[0]
    P, C = dec_b.shape
    E = edge_index.shape[1]

    NPAD = _round_up(N, LANES)
    E_pad = _round_up(E, 2 * CH)
    if NPAD == N and E_pad != E:
        NPAD += LANES
    x_p = jnp.pad(x, ((0, NPAD - N), (0, 0))) if NPAD != N else x
    if E_pad != E:
        pad = jnp.concatenate(
            [jnp.zeros((1, E_pad - E), jnp.int32),
             jnp.full((1, E_pad - E), N, jnp.int32)], axis=0)
        edges = jnp.concatenate([edge_index, pad], axis=1)
    else:
        edges = edge_index

    encoder = _make_encoder(NPAD, F, H)
    mlp = _make_mlp(NPAD, H)
    segmax = _make_segmax(NPAD, H, E_pad)
    decoder = _make_decoder(NPAD, H, C)

    hT = encoder(x_p, enc_W, enc_b[:, None])

    def body(_, hT):
        aggT = segmax(hT, edges)
        return mlp(hT, aggT, proc_W, proc_b[:, None])

    hT = lax.fori_loop(0, diameter, body, hT)

    o0, o1 = decoder(hT, dec_W[0], dec_b[0][:, None], dec_W[1], dec_b[1][:, None])
    return (o0.T[:N], o1.T[:N])
